# baseline (speedup 1.0000x reference)
import functools

import jax
import jax.numpy as jnp
from jax import lax
from jax.experimental import pallas as pl
from jax.experimental.pallas import tpu as pltpu
from jax.experimental.pallas import tpu_sc as plsc

NC = 2
NS = 16
NW = NC * NS

DP = 128
SUB = 128


def _emb_kernel(B, D, b0_tiles, u_per_w,
                idx_hbm, table_hbm, out_hbm,
                idx_v, g_a, g_b, t_a, t_b,
                gsem_a, gsem_b, ssem_a, ssem_b):
    wid = lax.axis_index("s") * NC + lax.axis_index("c")
    ubase = wid * u_per_w

    pltpu.sync_copy(idx_hbm.at[pl.ds(ubase * SUB, u_per_w * SUB)], idx_v)

    def fire(ul, gbuf, sem):
        pltpu.async_copy(
            table_hbm.at[idx_v.at[pl.ds(ul * SUB, SUB)]], gbuf, sem)

    def drain_gather(gbuf, sem):
        pltpu.make_async_copy(table_hbm.at[pl.ds(0, SUB)], gbuf, sem).wait()

    def drain_store(tbuf, sem):
        pltpu.make_async_copy(tbuf, out_hbm.at[0, :, 0], sem).wait()

    lane = lax.iota(jnp.int32, 16)
    b_vecs = [lane + 16 * k for k in range(SUB // 16)]
    d_vecs = [(lane + d) & 15 for d in range(16)]

    def transpose_store(ul, gbuf, tbuf, ssem):
        @pl.when(ul >= 2)
        def _():
            drain_store(tbuf, ssem)

        def eg_body(eg):
            ebase = 16 * eg
            for k in range(SUB // 16):
                for d in range(16):
                    e_vec = d_vecs[d] + ebase
                    v = plsc.load_gather(gbuf, [b_vecs[k], e_vec])
                    plsc.store_scatter(
                        tbuf, [e_vec >> 3, e_vec & 7, b_vecs[k]], v)

        pl.loop(0, D // 16)(eg_body)
        u = ubase + ul
        pltpu.async_copy(
            tbuf, out_hbm.at[u // b0_tiles, :, u % b0_tiles], ssem)

    fire(0, g_a, gsem_a)

    def pair_body(p):
        ua = 2 * p
        fire(ua + 1, g_b, gsem_b)
        drain_gather(g_a, gsem_a)
        transpose_store(ua, g_a, t_a, ssem_a)

        @pl.when(ua + 2 < u_per_w)
        def _():
            fire(ua + 2, g_a, gsem_a)

        drain_gather(g_b, gsem_b)
        transpose_store(ua + 1, g_b, t_b, ssem_b)

    pl.loop(0, u_per_w // 2)(pair_body)

    drain_store(t_a, ssem_a)
    drain_store(t_b, ssem_b)


def _make_emb(B, D):
    n_units = B // SUB
    assert n_units % NW == 0 and (n_units // NW) % 2 == 0
    u_per_w = n_units // NW
    b0_tiles = 4096 // SUB
    mesh = plsc.VectorSubcoreMesh(core_axis_name="c", subcore_axis_name="s")
    return pl.kernel(
        functools.partial(_emb_kernel, B, D, b0_tiles, u_per_w),
        out_type=jax.ShapeDtypeStruct(
            (B // 4096, D // 8, b0_tiles, 8, SUB), jnp.float32),
        mesh=mesh,
        scratch_types=[
            pltpu.VMEM((u_per_w * SUB,), jnp.int32),
            pltpu.VMEM((SUB, DP), jnp.float32),
            pltpu.VMEM((SUB, DP), jnp.float32),
            pltpu.VMEM((D // 8, 8, SUB), jnp.float32),
            pltpu.VMEM((D // 8, 8, SUB), jnp.float32),
            pltpu.SemaphoreType.DMA,
            pltpu.SemaphoreType.DMA,
            pltpu.SemaphoreType.DMA,
            pltpu.SemaphoreType.DMA,
        ],
        compiler_params=pltpu.CompilerParams(
            use_tc_tiling_on_sc=True, needs_layout_passes=False),
    )


@jax.jit
def kernel(input_tensor, W):
    B = input_tensor.size
    D = W.shape[1]
    idx_flat = input_tensor.reshape(B).astype(jnp.int32)
    Wp = jnp.pad(W, ((0, 0), (0, DP - D)))
    out5 = _make_emb(B, D)(idx_flat, Wp)
    out = out5.transpose(0, 2, 4, 1, 3).reshape(*input_tensor.shape, D)
    return out


# speedup vs baseline: 1.0665x; 1.0665x over previous
import functools

import jax
import jax.numpy as jnp
from jax import lax
from jax.experimental import pallas as pl
from jax.experimental.pallas import tpu as pltpu
from jax.experimental.pallas import tpu_sc as plsc

NC = 2
NS = 16
NW = NC * NS

DP = 128
SUB = 128


def _emb_kernel(B, D, b0_tiles, u_per_w,
                idx_hbm, table_hbm, out_hbm,
                idx_v, g_a, g_b, t_a, t_b,
                gsem_a, gsem_b, ssem_a, ssem_b):
    wid = lax.axis_index("s") * NC + lax.axis_index("c")
    ubase = wid * u_per_w

    pltpu.sync_copy(idx_hbm.at[pl.ds(ubase * SUB, u_per_w * SUB)], idx_v)

    def fire(ul, gbuf, sem):
        pltpu.async_copy(
            table_hbm.at[idx_v.at[pl.ds(ul * SUB, SUB)]], gbuf, sem)

    def drain_gather(gbuf, sem):
        pltpu.make_async_copy(table_hbm.at[pl.ds(0, SUB)], gbuf, sem).wait()

    def drain_store(tbuf, sem):
        for eh in range(D // 8):
            pltpu.make_async_copy(
                tbuf.at[pl.ds(8 * eh, 8)], out_hbm.at[0, eh, 0], sem).wait()

    lane = lax.iota(jnp.int32, 16)
    d_vecs = [(lane + d) & 15 for d in range(16)]
    b_vecs = [lane + 16 * k for k in range(SUB // 16)]

    def transpose_store(ul, gbuf, tbuf, ssem):
        @pl.when(ul >= 2)
        def _():
            drain_store(tbuf, ssem)

        def eg_body(eg):
            ebase = 16 * eg
            e_vecs = [d_vecs[d] + ebase for d in range(16)]
            for k in range(SUB // 16):
                for d in range(16):
                    v = plsc.load_gather(gbuf, [b_vecs[k], e_vecs[d]])
                    plsc.store_scatter(tbuf, [e_vecs[d], b_vecs[k]], v)

        pl.loop(0, D // 16)(eg_body)
        u = ubase + ul
        b0 = u // b0_tiles
        b1h = u % b0_tiles
        for eh in range(D // 8):
            pltpu.async_copy(
                tbuf.at[pl.ds(8 * eh, 8)], out_hbm.at[b0, eh, b1h], ssem)

    fire(0, g_a, gsem_a)

    def pair_body(p):
        ua = 2 * p
        fire(ua + 1, g_b, gsem_b)
        drain_gather(g_a, gsem_a)
        transpose_store(ua, g_a, t_a, ssem_a)

        @pl.when(ua + 2 < u_per_w)
        def _():
            fire(ua + 2, g_a, gsem_a)

        drain_gather(g_b, gsem_b)
        transpose_store(ua + 1, g_b, t_b, ssem_b)

    pl.loop(0, u_per_w // 2)(pair_body)

    drain_store(t_a, ssem_a)
    drain_store(t_b, ssem_b)


def _make_emb(B, D):
    n_units = B // SUB
    assert n_units % NW == 0 and (n_units // NW) % 2 == 0
    u_per_w = n_units // NW
    b0_tiles = 4096 // SUB
    mesh = plsc.VectorSubcoreMesh(core_axis_name="c", subcore_axis_name="s")
    return pl.kernel(
        functools.partial(_emb_kernel, B, D, b0_tiles, u_per_w),
        out_type=jax.ShapeDtypeStruct(
            (B // 4096, D // 8, b0_tiles, 8, SUB), jnp.float32),
        mesh=mesh,
        scratch_types=[
            pltpu.VMEM((u_per_w * SUB,), jnp.int32),
            pltpu.VMEM((SUB, DP), jnp.float32),
            pltpu.VMEM((SUB, DP), jnp.float32),
            pltpu.VMEM((D, SUB), jnp.float32),
            pltpu.VMEM((D, SUB), jnp.float32),
            pltpu.SemaphoreType.DMA,
            pltpu.SemaphoreType.DMA,
            pltpu.SemaphoreType.DMA,
            pltpu.SemaphoreType.DMA,
        ],
        compiler_params=pltpu.CompilerParams(
            use_tc_tiling_on_sc=True, needs_layout_passes=False),
    )


@jax.jit
def kernel(input_tensor, W):
    B = input_tensor.size
    D = W.shape[1]
    idx_flat = input_tensor.reshape(B).astype(jnp.int32)
    Wp = jnp.pad(W, ((0, 0), (0, DP - D)))
    out5 = _make_emb(B, D)(idx_flat, Wp)
    out = out5.transpose(0, 2, 4, 1, 3).reshape(*input_tensor.shape, D)
    return out


# speedup vs baseline: 1.2257x; 1.1493x over previous
import functools

import jax
import jax.numpy as jnp
from jax import lax
from jax.experimental import pallas as pl
from jax.experimental.pallas import tpu as pltpu
from jax.experimental.pallas import tpu_sc as plsc

NC = 2
NS = 16
NW = NC * NS

DP = 128
SUB = 128


def _emb_kernel(B, D, b0_tiles, u_per_w,
                idx_hbm, table_hbm, out_hbm,
                idx_v, g_a, g_b, t_a, t_b,
                gsem_a, gsem_b, ssem_a, ssem_b):
    wid = lax.axis_index("s") * NC + lax.axis_index("c")
    ubase = wid * u_per_w

    pltpu.sync_copy(idx_hbm.at[pl.ds(ubase * SUB, u_per_w * SUB)], idx_v)

    def fire(ul, gbuf, sem):
        pltpu.async_copy(
            table_hbm.at[idx_v.at[pl.ds(ul * SUB, SUB)]], gbuf, sem)

    def drain_gather(gbuf, sem):
        pltpu.make_async_copy(table_hbm.at[pl.ds(0, SUB)], gbuf, sem).wait()

    def drain_store(tbuf, sem):
        for eh in range(D // 8):
            pltpu.make_async_copy(
                tbuf.at[pl.ds(8 * eh, 8)], out_hbm.at[0, eh, 0], sem).wait()

    lane = lax.iota(jnp.int32, 16)
    d_vecs = [(lane + d) & 15 for d in range(16)]
    b_vecs = [lane + 16 * k for k in range(SUB // 16)]

    def transpose_store(ul, gbuf, tbuf, ssem):
        @pl.when(ul >= 2)
        def _():
            drain_store(tbuf, ssem)

        def eg_body(eg):
            ebase = 16 * eg
            e_vecs = [d_vecs[d] + ebase for d in range(16)]
            for k in range(SUB // 16):
                vs = [plsc.load_gather(gbuf, [b_vecs[k], e_vecs[d]])
                      for d in range(16)]
                for d in range(16):
                    plsc.store_scatter(tbuf, [e_vecs[d], b_vecs[k]], vs[d])

        pl.loop(0, D // 16)(eg_body)
        u = ubase + ul
        b0 = u // b0_tiles
        b1h = u % b0_tiles
        for eh in range(D // 8):
            pltpu.async_copy(
                tbuf.at[pl.ds(8 * eh, 8)], out_hbm.at[b0, eh, b1h], ssem)

    fire(0, g_a, gsem_a)

    def pair_body(p):
        ua = 2 * p
        fire(ua + 1, g_b, gsem_b)
        drain_gather(g_a, gsem_a)
        transpose_store(ua, g_a, t_a, ssem_a)

        @pl.when(ua + 2 < u_per_w)
        def _():
            fire(ua + 2, g_a, gsem_a)

        drain_gather(g_b, gsem_b)
        transpose_store(ua + 1, g_b, t_b, ssem_b)

    pl.loop(0, u_per_w // 2)(pair_body)

    drain_store(t_a, ssem_a)
    drain_store(t_b, ssem_b)


def _make_emb(B, D):
    n_units = B // SUB
    assert n_units % NW == 0 and (n_units // NW) % 2 == 0
    u_per_w = n_units // NW
    b0_tiles = 4096 // SUB
    mesh = plsc.VectorSubcoreMesh(core_axis_name="c", subcore_axis_name="s")
    return pl.kernel(
        functools.partial(_emb_kernel, B, D, b0_tiles, u_per_w),
        out_type=jax.ShapeDtypeStruct(
            (B // 4096, D // 8, b0_tiles, 8, SUB), jnp.float32),
        mesh=mesh,
        scratch_types=[
            pltpu.VMEM((u_per_w * SUB,), jnp.int32),
            pltpu.VMEM((SUB, DP), jnp.float32),
            pltpu.VMEM((SUB, DP), jnp.float32),
            pltpu.VMEM((D, SUB), jnp.float32),
            pltpu.VMEM((D, SUB), jnp.float32),
            pltpu.SemaphoreType.DMA,
            pltpu.SemaphoreType.DMA,
            pltpu.SemaphoreType.DMA,
            pltpu.SemaphoreType.DMA,
        ],
        compiler_params=pltpu.CompilerParams(
            use_tc_tiling_on_sc=True, needs_layout_passes=False),
    )


@jax.jit
def kernel(input_tensor, W):
    B = input_tensor.size
    D = W.shape[1]
    idx_flat = input_tensor.reshape(B).astype(jnp.int32)
    Wp = jnp.pad(W, ((0, 0), (0, DP - D)))
    out5 = _make_emb(B, D)(idx_flat, Wp)
    out = out5.transpose(0, 2, 4, 1, 3).reshape(*input_tensor.shape, D)
    return out


# speedup vs baseline: 1.4446x; 1.1786x over previous
import functools

import jax
import jax.numpy as jnp
from jax import lax
from jax.experimental import pallas as pl
from jax.experimental.pallas import tpu as pltpu
from jax.experimental.pallas import tpu_sc as plsc

NC = 2
NS = 16
NW = NC * NS

DP = 128
SUB = 128


def _emb_kernel(B, D, b0_tiles, u_per_w,
                idx_hbm, table_hbm, out_hbm,
                idx_v, g_a, g_b, t_a, t_b,
                gsem_a, gsem_b, ssem_a, ssem_b):
    wid = lax.axis_index("s") * NC + lax.axis_index("c")
    ubase = wid * u_per_w

    pltpu.sync_copy(idx_hbm.at[pl.ds(ubase * SUB, u_per_w * SUB)], idx_v)

    def fire(ul, gbuf, sem):
        pltpu.async_copy(
            table_hbm.at[idx_v.at[pl.ds(ul * SUB, SUB)]], gbuf, sem)

    def drain_gather(gbuf, sem):
        pltpu.make_async_copy(table_hbm.at[pl.ds(0, SUB)], gbuf, sem).wait()

    def drain_store(tbuf, sem):
        for eh in range(D // 8):
            pltpu.make_async_copy(
                tbuf.at[pl.ds(8 * eh, 8)], out_hbm.at[0, eh, 0], sem).wait()

    lane = lax.iota(jnp.int32, 16)
    d_vecs = [(lane + d) & 15 for d in range(16)]
    b_vecs = [lane + 16 * k for k in range(SUB // 16)]

    def transpose_store(ul, gbuf, tbuf, ssem):
        @pl.when(ul >= 2)
        def _():
            drain_store(tbuf, ssem)

        def eg_body(eg):
            ebase = 16 * eg
            e_vecs = [d_vecs[d] + ebase for d in range(16)]
            for k in range(SUB // 16):
                vs = [plsc.load_gather(gbuf, [b_vecs[k], e_vecs[d]])
                      for d in range(16)]
                for d in range(16):
                    plsc.store_scatter(tbuf, [e_vecs[d], b_vecs[k]], vs[d])

        pl.loop(0, D // 16)(eg_body)
        u = ubase + ul
        b0 = u // b0_tiles
        b1h = u % b0_tiles
        for eh in range(D // 8):
            pltpu.async_copy(
                tbuf.at[pl.ds(8 * eh, 8)], out_hbm.at[b0, eh, b1h], ssem)

    fire(0, g_a, gsem_a)

    def pair_body(p):
        ua = 2 * p
        fire(ua + 1, g_b, gsem_b)
        drain_gather(g_a, gsem_a)
        transpose_store(ua, g_a, t_a, ssem_a)

        @pl.when(ua + 2 < u_per_w)
        def _():
            fire(ua + 2, g_a, gsem_a)

        drain_gather(g_b, gsem_b)
        transpose_store(ua + 1, g_b, t_b, ssem_b)

    pl.loop(0, u_per_w // 2)(pair_body)

    drain_store(t_a, ssem_a)
    drain_store(t_b, ssem_b)


V = 1000000
VFULL = (V // SUB) * SUB
NFC = V // SUB


def _fmt_kernel(Wt_hbm, tail_hbm, s_hbm,
                in_a, in_b, out_a, out_b, tail_v,
                isem_a, isem_b, osem_a, osem_b):
    D = Wt_hbm.shape[0]
    wid = lax.axis_index("s") * NC + lax.axis_index("c")
    extra = NFC - 32 * (NFC // 32)
    cnt = (NFC // 32) + jnp.where(wid < extra, 1, 0)
    cstart = (NFC // 32) * wid + jnp.minimum(wid, extra)

    isems = [isem_a, isem_b]
    osems = [osem_a, osem_b]
    ins = [in_a, in_b]
    outs = [out_a, out_b]

    def fire(cl, s):
        pltpu.async_copy(
            Wt_hbm.at[:, pl.ds((cstart + cl) * SUB, SUB)], ins[s], isems[s])

    def drain_in(s):
        pltpu.make_async_copy(Wt_hbm.at[:, pl.ds(0, SUB)], ins[s],
                              isems[s]).wait()

    def drain_out(s):
        pltpu.make_async_copy(outs[s], s_hbm.at[pl.ds(0, SUB)],
                              osems[s]).wait()

    lane = lax.iota(jnp.int32, 16)
    d_vecs = [(lane + d) & 15 for d in range(16)]

    def chunk_body(cl):
        for s in range(2):
            @pl.when((cl & 1) == s)
            def _():
                @pl.when(cl >= 2)
                def _():
                    drain_out(s)

                drain_in(s)

                def vg_body(vg):
                    vvec = lane + 16 * vg
                    for g in range(D // 16):
                        evs = [d_vecs[d] + 16 * g for d in range(16)]
                        vs = [plsc.load_gather(ins[s], [evs[d], vvec])
                              for d in range(16)]
                        for d in range(16):
                            plsc.store_scatter(outs[s], [vvec, evs[d]], vs[d])

                pl.loop(0, SUB // 16)(vg_body)
                pltpu.async_copy(
                    outs[s], s_hbm.at[pl.ds((cstart + cl) * SUB, SUB)],
                    osems[s])

                @pl.when(cl + 2 < cnt)
                def _():
                    fire(cl + 2, s)

    fire(0, 0)
    fire(1, 1)
    pl.loop(0, cnt)(chunk_body)
    drain_out(0)
    drain_out(1)

    @pl.when(wid == NW - 1)
    def _():
        pltpu.sync_copy(tail_hbm, tail_v)
        pltpu.sync_copy(tail_v, s_hbm.at[pl.ds(VFULL, V - VFULL)])


def _make_fmt(D):
    mesh = plsc.VectorSubcoreMesh(core_axis_name="c", subcore_axis_name="s")
    return pl.kernel(
        _fmt_kernel,
        out_type=jax.ShapeDtypeStruct((V, DP), jnp.float32),
        mesh=mesh,
        scratch_types=[
            pltpu.VMEM((D, SUB), jnp.float32),
            pltpu.VMEM((D, SUB), jnp.float32),
            pltpu.VMEM((SUB, DP), jnp.float32),
            pltpu.VMEM((SUB, DP), jnp.float32),
            pltpu.VMEM((V - VFULL, DP), jnp.float32),
            pltpu.SemaphoreType.DMA,
            pltpu.SemaphoreType.DMA,
            pltpu.SemaphoreType.DMA,
            pltpu.SemaphoreType.DMA,
        ],
        compiler_params=pltpu.CompilerParams(
            use_tc_tiling_on_sc=True, needs_layout_passes=False),
    )


def _make_emb(B, D):
    n_units = B // SUB
    assert n_units % NW == 0 and (n_units // NW) % 2 == 0
    u_per_w = n_units // NW
    b0_tiles = 4096 // SUB
    mesh = plsc.VectorSubcoreMesh(core_axis_name="c", subcore_axis_name="s")
    return pl.kernel(
        functools.partial(_emb_kernel, B, D, b0_tiles, u_per_w),
        out_type=jax.ShapeDtypeStruct(
            (B // 4096, D // 8, b0_tiles, 8, SUB), jnp.float32),
        mesh=mesh,
        scratch_types=[
            pltpu.VMEM((u_per_w * SUB,), jnp.int32),
            pltpu.VMEM((SUB, DP), jnp.float32),
            pltpu.VMEM((SUB, DP), jnp.float32),
            pltpu.VMEM((D, SUB), jnp.float32),
            pltpu.VMEM((D, SUB), jnp.float32),
            pltpu.SemaphoreType.DMA,
            pltpu.SemaphoreType.DMA,
            pltpu.SemaphoreType.DMA,
            pltpu.SemaphoreType.DMA,
        ],
        compiler_params=pltpu.CompilerParams(
            use_tc_tiling_on_sc=True, needs_layout_passes=False),
    )


@jax.jit
def kernel(input_tensor, W):
    B = input_tensor.size
    D = W.shape[1]
    idx_flat = input_tensor.reshape(B).astype(jnp.int32)
    tail = jnp.pad(W[VFULL:], ((0, 0), (0, DP - D)))
    S = _make_fmt(D)(W.T, tail)
    out5 = _make_emb(B, D)(idx_flat, S)
    out = out5.transpose(0, 2, 4, 1, 3).reshape(*input_tensor.shape, D)
    return out


# speedup vs baseline: 1.8773x; 1.2996x over previous
import functools

import jax
import jax.numpy as jnp
from jax import lax
from jax.experimental import pallas as pl
from jax.experimental.pallas import tpu as pltpu
from jax.experimental.pallas import tpu_sc as plsc

NC = 2
NS = 16
NW = NC * NS

DP = 128
SUB = 128


def _emb_kernel(B, D, b0_tiles, u_per_w,
                idx_hbm, table_hbm, out_hbm,
                idx_v, idx2_v, g_a, g_b, t_a, t_b,
                gsem_a, gsem_b, ssem_a, ssem_b):
    wid = lax.axis_index("s") * NC + lax.axis_index("c")
    ubase = wid * u_per_w

    pltpu.sync_copy(idx_hbm.at[pl.ds(ubase * SUB, u_per_w * SUB)], idx_v)

    def shift_body(c):
        for k in range(SUB // 16):
            o = c * SUB + 16 * k
            idx2_v[pl.ds(o, 16)] = idx_v[pl.ds(o, 16)] >> 1

    pl.loop(0, u_per_w)(shift_body)

    def fire(ul, gbuf, sem):
        pltpu.async_copy(
            table_hbm.at[idx2_v.at[pl.ds(ul * SUB, SUB)]], gbuf, sem)

    def drain_gather(gbuf, sem):
        pltpu.make_async_copy(table_hbm.at[pl.ds(0, SUB)], gbuf, sem).wait()

    def drain_store(tbuf, sem):
        for eh in range(D // 8):
            pltpu.make_async_copy(
                tbuf.at[pl.ds(8 * eh, 8)], out_hbm.at[0, eh, 0], sem).wait()

    lane = lax.iota(jnp.int32, 16)
    d_vecs = [(lane + d) & 15 for d in range(16)]
    b_vecs = [lane + 16 * k for k in range(SUB // 16)]

    def transpose_store(ul, gbuf, tbuf, ssem):
        @pl.when(ul >= 2)
        def _():
            drain_store(tbuf, ssem)

        pvecs = [((idx_v[pl.ds(ul * SUB + 16 * k, 16)]) & 1) << 6
                 for k in range(SUB // 16)]

        def eg_body(eg):
            ebase = 16 * eg
            e_vecs = [d_vecs[d] + ebase for d in range(16)]
            for k in range(SUB // 16):
                vs = [plsc.load_gather(gbuf,
                                       [b_vecs[k], e_vecs[d] + pvecs[k]])
                      for d in range(16)]
                for d in range(16):
                    plsc.store_scatter(tbuf, [e_vecs[d], b_vecs[k]], vs[d])

        pl.loop(0, D // 16)(eg_body)
        u = ubase + ul
        b0 = u // b0_tiles
        b1h = u % b0_tiles
        for eh in range(D // 8):
            pltpu.async_copy(
                tbuf.at[pl.ds(8 * eh, 8)], out_hbm.at[b0, eh, b1h], ssem)

    fire(0, g_a, gsem_a)

    def pair_body(p):
        ua = 2 * p
        fire(ua + 1, g_b, gsem_b)
        drain_gather(g_a, gsem_a)
        transpose_store(ua, g_a, t_a, ssem_a)

        @pl.when(ua + 2 < u_per_w)
        def _():
            fire(ua + 2, g_a, gsem_a)

        drain_gather(g_b, gsem_b)
        transpose_store(ua + 1, g_b, t_b, ssem_b)

    pl.loop(0, u_per_w // 2)(pair_body)

    drain_store(t_a, ssem_a)
    drain_store(t_b, ssem_b)


V = 1000000
VFULL = (V // SUB) * SUB
NFC = V // SUB


def _fmt_kernel(Wt_hbm, tail_hbm, s_hbm,
                in_a, in_b, out_a, out_b, tail_v,
                isem_a, isem_b, osem_a, osem_b):
    D = Wt_hbm.shape[0]
    wid = lax.axis_index("s") * NC + lax.axis_index("c")
    extra = NFC - 32 * (NFC // 32)
    cnt = (NFC // 32) + jnp.where(wid < extra, 1, 0)
    cstart = (NFC // 32) * wid + jnp.minimum(wid, extra)

    isems = [isem_a, isem_b]
    osems = [osem_a, osem_b]
    ins = [in_a, in_b]
    outs = [out_a, out_b]
    HSUB = SUB // 2

    def fire(cl, s):
        pltpu.async_copy(
            Wt_hbm.at[:, pl.ds((cstart + cl) * SUB, SUB)], ins[s], isems[s])

    def drain_in(s):
        pltpu.make_async_copy(Wt_hbm.at[:, pl.ds(0, SUB)], ins[s],
                              isems[s]).wait()

    def drain_out(s):
        pltpu.make_async_copy(outs[s], s_hbm.at[pl.ds(0, HSUB)],
                              osems[s]).wait()

    lane = lax.iota(jnp.int32, 16)
    d_vecs = [(lane + d) & 15 for d in range(16)]

    def chunk_body(cl):
        for s in range(2):
            @pl.when((cl & 1) == s)
            def _():
                @pl.when(cl >= 2)
                def _():
                    drain_out(s)

                drain_in(s)

                def vg_body(vg):
                    vvec = lane + 16 * vg
                    pvec = (vvec >> 1)
                    cbase = (vvec & 1) << 6
                    for g in range(D // 16):
                        evs = [d_vecs[d] + 16 * g for d in range(16)]
                        vs = [plsc.load_gather(ins[s], [evs[d], vvec])
                              for d in range(16)]
                        for d in range(16):
                            plsc.store_scatter(
                                outs[s], [pvec, cbase + evs[d]], vs[d])

                pl.loop(0, SUB // 16)(vg_body)
                pltpu.async_copy(
                    outs[s], s_hbm.at[pl.ds((cstart + cl) * HSUB, HSUB)],
                    osems[s])

                @pl.when(cl + 2 < cnt)
                def _():
                    fire(cl + 2, s)

    fire(0, 0)
    fire(1, 1)
    pl.loop(0, cnt)(chunk_body)
    drain_out(0)
    drain_out(1)

    @pl.when(wid == NW - 1)
    def _():
        pltpu.sync_copy(tail_hbm, tail_v)
        pltpu.sync_copy(
            tail_v, s_hbm.at[pl.ds(VFULL // 2, (V - VFULL) // 2)])


def _make_fmt(D):
    mesh = plsc.VectorSubcoreMesh(core_axis_name="c", subcore_axis_name="s")
    return pl.kernel(
        _fmt_kernel,
        out_type=jax.ShapeDtypeStruct((V // 2, DP), jnp.float32),
        mesh=mesh,
        scratch_types=[
            pltpu.VMEM((D, SUB), jnp.float32),
            pltpu.VMEM((D, SUB), jnp.float32),
            pltpu.VMEM((SUB // 2, DP), jnp.float32),
            pltpu.VMEM((SUB // 2, DP), jnp.float32),
            pltpu.VMEM(((V - VFULL) // 2, DP), jnp.float32),
            pltpu.SemaphoreType.DMA,
            pltpu.SemaphoreType.DMA,
            pltpu.SemaphoreType.DMA,
            pltpu.SemaphoreType.DMA,
        ],
        compiler_params=pltpu.CompilerParams(
            use_tc_tiling_on_sc=True, needs_layout_passes=False),
    )


def _make_emb(B, D):
    n_units = B // SUB
    assert n_units % NW == 0 and (n_units // NW) % 2 == 0
    u_per_w = n_units // NW
    b0_tiles = 4096 // SUB
    mesh = plsc.VectorSubcoreMesh(core_axis_name="c", subcore_axis_name="s")
    return pl.kernel(
        functools.partial(_emb_kernel, B, D, b0_tiles, u_per_w),
        out_type=jax.ShapeDtypeStruct(
            (B // 4096, D // 8, b0_tiles, 8, SUB), jnp.float32),
        mesh=mesh,
        scratch_types=[
            pltpu.VMEM((u_per_w * SUB,), jnp.int32),
            pltpu.VMEM((u_per_w * SUB,), jnp.int32),
            pltpu.VMEM((SUB, DP), jnp.float32),
            pltpu.VMEM((SUB, DP), jnp.float32),
            pltpu.VMEM((D, SUB), jnp.float32),
            pltpu.VMEM((D, SUB), jnp.float32),
            pltpu.SemaphoreType.DMA,
            pltpu.SemaphoreType.DMA,
            pltpu.SemaphoreType.DMA,
            pltpu.SemaphoreType.DMA,
        ],
        compiler_params=pltpu.CompilerParams(
            use_tc_tiling_on_sc=True, needs_layout_passes=False),
    )


@jax.jit
def kernel(input_tensor, W):
    B = input_tensor.size
    D = W.shape[1]
    idx_flat = input_tensor.reshape(B).astype(jnp.int32)
    tail = W[VFULL:].reshape((V - VFULL) // 2, DP)
    S = _make_fmt(D)(W.T, tail)
    out5 = _make_emb(B, D)(idx_flat, S)
    out = out5.transpose(0, 2, 4, 1, 3).reshape(*input_tensor.shape, D)
    return out
